# Initial kernel scaffold; baseline (speedup 1.0000x reference)
import jax
import jax.numpy as jnp
from jax.experimental import pallas as pl


def kernel(indices, weights):
    raise NotImplementedError("write your pallas kernel here")


# speedup vs baseline: 2.7393x; 2.7393x over previous
import functools

import jax
import jax.numpy as jnp
from jax import lax
from jax.experimental import pallas as pl
from jax.experimental.pallas import tpu as pltpu
from jax.experimental.pallas import tpu_sc as plsc


def _make_gather(nw, nc, nchunks, chunk, D):
    B = nw * nchunks * chunk
    mesh = plsc.VectorSubcoreMesh(core_axis_name="c", subcore_axis_name="s")

    @functools.partial(
        pl.kernel,
        mesh=mesh,
        out_type=jax.ShapeDtypeStruct((B, D), jnp.float32),
        scratch_types=[
            pltpu.VMEM((nchunks, chunk), jnp.int32),
            pltpu.VMEM((chunk, D), jnp.float32),
            pltpu.SemaphoreType.DMA,
        ],
    )
    def k(idx_hbm, table_hbm, out_hbm, idx_v, rows_v, sem):
        wid = lax.axis_index("s") * nc + lax.axis_index("c")
        base = wid * (nchunks * chunk)
        pltpu.sync_copy(idx_hbm.at[wid], idx_v)

        def body(j, carry):
            pltpu.async_copy(table_hbm.at[idx_v.at[j]], rows_v, sem).wait()
            pltpu.sync_copy(rows_v, out_hbm.at[pl.ds(base + j * chunk, chunk)])
            return carry

        lax.fori_loop(0, nchunks, body, 0)

    return k


def kernel(indices, weights):
    bsz, seq = indices.shape
    D = weights.shape[1]
    B = bsz * seq
    info = plsc.get_sparse_core_info()
    nc, ns = info.num_cores, info.num_subcores
    nw = nc * ns
    chunk = 128
    flat = indices.astype(jnp.int32).reshape(-1)
    pad = (-B) % (nw * chunk)
    if pad:
        flat = jnp.concatenate([flat, jnp.zeros((pad,), jnp.int32)])
    nchunks = (B + pad) // (nw * chunk)
    idx3 = flat.reshape(nw, nchunks, chunk)
    out = _make_gather(nw, nc, nchunks, chunk, D)(idx3, weights)
    return out[:B].reshape(bsz, seq, D)


# speedup vs baseline: 2.8249x; 1.0312x over previous
import functools

import jax
import jax.numpy as jnp
from jax import lax
from jax.experimental import pallas as pl
from jax.experimental.pallas import tpu as pltpu
from jax.experimental.pallas import tpu_sc as plsc


def _make_gather(nw, nc, nchunks, chunk, D):
    B = nw * nchunks * chunk
    rows_per_w = nchunks * chunk
    mesh = plsc.VectorSubcoreMesh(core_axis_name="c", subcore_axis_name="s")

    @functools.partial(
        pl.kernel,
        mesh=mesh,
        out_type=jax.ShapeDtypeStruct((B, D), jnp.float32),
        scratch_types=[
            pltpu.VMEM((nchunks, chunk), jnp.int32),
            pltpu.VMEM((2, chunk, D), jnp.float32),
            pltpu.SemaphoreType.DMA,
            pltpu.SemaphoreType.DMA,
            pltpu.SemaphoreType.DMA,
            pltpu.SemaphoreType.DMA,
        ],
    )
    def k(idx_hbm, table_hbm, out_hbm, idx_v, rows_v, g0, g1, w0, w1):
        wid = lax.axis_index("s") * nc + lax.axis_index("c")
        base = wid * rows_per_w
        pltpu.sync_copy(idx_hbm.at[wid], idx_v)
        buf = (rows_v.at[0], rows_v.at[1])
        gsem = (g0, g1)
        wsem = (w0, w1)

        def gather(j, p, sem):
            return pltpu.make_async_copy(table_hbm.at[idx_v.at[j]], buf[p], sem)

        def write(j, p, sem):
            return pltpu.make_async_copy(
                buf[p], out_hbm.at[pl.ds(base + j * chunk, chunk)], sem)

        gather(0, 0, g0).start()
        gather(1, 1, g1).start()

        def body(g, carry):
            j2 = 2 * g
            for p in range(2):
                j = j2 + p
                gather(j, p, gsem[p]).wait()
                wd = write(j, p, wsem[p])
                wd.start()
                wd.wait()
                nxt = j + 2

                @pl.when(nxt < nchunks)
                def _():
                    gather(nxt, p, gsem[p]).start()
            return carry

        lax.fori_loop(0, nchunks // 2, body, 0)

    return k


def kernel(indices, weights):
    bsz, seq = indices.shape
    D = weights.shape[1]
    B = bsz * seq
    info = plsc.get_sparse_core_info()
    nc, ns = info.num_cores, info.num_subcores
    nw = nc * ns
    chunk = 128
    flat = indices.astype(jnp.int32).reshape(-1)
    pad = (-B) % (nw * chunk * 2)
    if pad:
        flat = jnp.concatenate([flat, jnp.zeros((pad,), jnp.int32)])
    nchunks = (B + pad) // (nw * chunk)
    idx3 = flat.reshape(nw, nchunks, chunk)
    out = _make_gather(nw, nc, nchunks, chunk, D)(idx3, weights)
    return out[:B].reshape(bsz, seq, D)


# speedup vs baseline: 3.6881x; 1.3056x over previous
import functools

import jax
import jax.numpy as jnp
from jax import lax
from jax.experimental import pallas as pl
from jax.experimental.pallas import tpu as pltpu
from jax.experimental.pallas import tpu_sc as plsc


def _make_gather(nw, nc, nchunks, chunk, D, V):
    B = nw * nchunks * chunk
    rows_per_w = nchunks * chunk
    ns = nw // nc
    v_per_tile = V // ns
    mesh = plsc.VectorSubcoreMesh(core_axis_name="c", subcore_axis_name="s")

    @functools.partial(
        pl.kernel,
        mesh=mesh,
        out_type=jax.ShapeDtypeStruct((B, D), jnp.float32),
        scratch_types=[
            pltpu.VMEM((nchunks, chunk), jnp.int32),
            pltpu.VMEM((2, chunk, D), jnp.float32),
            pltpu.VMEM_SHARED((V, D), jnp.float32),
            pltpu.SemaphoreType.DMA,
            pltpu.SemaphoreType.DMA,
            pltpu.SemaphoreType.DMA,
            pltpu.SemaphoreType.DMA,
        ],
    )
    def k(idx_hbm, table_hbm, out_hbm, idx_v, rows_v, table_sh, g0, g1, w0, w1):
        sid = lax.axis_index("s")
        wid = sid * nc + lax.axis_index("c")
        base = wid * rows_per_w
        pltpu.sync_copy(table_hbm.at[pl.ds(sid * v_per_tile, v_per_tile)],
                        table_sh.at[pl.ds(sid * v_per_tile, v_per_tile)])
        pltpu.sync_copy(idx_hbm.at[wid], idx_v)
        plsc.subcore_barrier()
        buf = (rows_v.at[0], rows_v.at[1])
        gsem = (g0, g1)
        wsem = (w0, w1)

        def gather(j, p, sem):
            return pltpu.make_async_copy(table_sh.at[idx_v.at[j]], buf[p], sem)

        def write(j, p, sem):
            return pltpu.make_async_copy(
                buf[p], out_hbm.at[pl.ds(base + j * chunk, chunk)], sem)

        gather(0, 0, g0).start()
        gather(1, 1, g1).start()

        def body(g, carry):
            j2 = 2 * g
            for p in range(2):
                j = j2 + p
                gather(j, p, gsem[p]).wait()
                wd = write(j, p, wsem[p])
                wd.start()
                wd.wait()
                nxt = j + 2

                @pl.when(nxt < nchunks)
                def _():
                    gather(nxt, p, gsem[p]).start()
            return carry

        lax.fori_loop(0, nchunks // 2, body, 0)

    return k


def kernel(indices, weights):
    bsz, seq = indices.shape
    D = weights.shape[1]
    B = bsz * seq
    info = plsc.get_sparse_core_info()
    nc, ns = info.num_cores, info.num_subcores
    nw = nc * ns
    chunk = 128
    flat = indices.astype(jnp.int32).reshape(-1)
    pad = (-B) % (nw * chunk * 2)
    if pad:
        flat = jnp.concatenate([flat, jnp.zeros((pad,), jnp.int32)])
    nchunks = (B + pad) // (nw * chunk)
    idx3 = flat.reshape(nw, nchunks, chunk)
    out = _make_gather(nw, nc, nchunks, chunk, D, weights.shape[0])(idx3, weights)
    return out[:B].reshape(bsz, seq, D)


# speedup vs baseline: 3.7082x; 1.0054x over previous
import functools

import jax
import jax.numpy as jnp
from jax import lax
from jax.experimental import pallas as pl
from jax.experimental.pallas import tpu as pltpu
from jax.experimental.pallas import tpu_sc as plsc


def _make_gather(nw, nc, nchunks, chunk, D, V):
    B = nw * nchunks * chunk
    rows_per_w = nchunks * chunk
    ns = nw // nc
    v_per_tile = V // ns
    mesh = plsc.VectorSubcoreMesh(core_axis_name="c", subcore_axis_name="s")

    @functools.partial(
        pl.kernel,
        mesh=mesh,
        out_type=jax.ShapeDtypeStruct((B, D), jnp.float32),
        scratch_types=[
            pltpu.VMEM((nchunks, chunk), jnp.int32),
            pltpu.VMEM((4, chunk, D), jnp.float32),
            pltpu.VMEM_SHARED((V, D), jnp.float32),
            pltpu.SemaphoreType.DMA((4,)),
            pltpu.SemaphoreType.DMA((4,)),
        ],
    )
    def k(idx_hbm, table_hbm, out_hbm, idx_v, rows_v, table_sh, gsem, wsem):
        sid = lax.axis_index("s")
        wid = sid * nc + lax.axis_index("c")
        base = wid * rows_per_w
        pltpu.sync_copy(table_hbm.at[pl.ds(sid * v_per_tile, v_per_tile)],
                        table_sh.at[pl.ds(sid * v_per_tile, v_per_tile)])
        pltpu.sync_copy(idx_hbm.at[wid], idx_v)
        plsc.subcore_barrier()

        def gather(j, p):
            return pltpu.make_async_copy(
                table_sh.at[idx_v.at[j]], rows_v.at[p], gsem.at[p])

        def write(j, p):
            return pltpu.make_async_copy(
                rows_v.at[p], out_hbm.at[pl.ds(base + j * chunk, chunk)],
                wsem.at[p])

        gather(0, 0).start()
        gather(1, 1).start()

        def body(j, carry):
            p = j % 4
            q = (j + 2) % 4

            @pl.when(j >= 2)
            def _():
                write(j - 2, q).wait()

            @pl.when(j + 2 < nchunks)
            def _():
                gather(j + 2, q).start()

            gather(j, p).wait()
            write(j, p).start()
            return carry

        lax.fori_loop(0, nchunks, body, 0)
        write(nchunks - 2, (nchunks - 2) % 4).wait()
        write(nchunks - 1, (nchunks - 1) % 4).wait()

    return k


def kernel(indices, weights):
    bsz, seq = indices.shape
    D = weights.shape[1]
    B = bsz * seq
    info = plsc.get_sparse_core_info()
    nc, ns = info.num_cores, info.num_subcores
    nw = nc * ns
    chunk = 128
    flat = indices.astype(jnp.int32).reshape(-1)
    pad = (-B) % (nw * chunk * 2)
    if pad:
        flat = jnp.concatenate([flat, jnp.zeros((pad,), jnp.int32)])
    nchunks = (B + pad) // (nw * chunk)
    idx3 = flat.reshape(nw, nchunks, chunk)
    out = _make_gather(nw, nc, nchunks, chunk, D, weights.shape[0])(idx3, weights)
    return out[:B].reshape(bsz, seq, D)


# speedup vs baseline: 15.8704x; 4.2798x over previous
import functools

import jax
import jax.numpy as jnp
from jax import lax
from jax.experimental import pallas as pl
from jax.experimental.pallas import tpu as pltpu
from jax.experimental.pallas import tpu_sc as plsc


def _make_gather(nw, nc, seq, bpw, D, V, bsz):
    ns = nw // nc
    v_per_tile = V // ns
    mesh = plsc.VectorSubcoreMesh(core_axis_name="c", subcore_axis_name="s")

    @functools.partial(
        pl.kernel,
        mesh=mesh,
        out_type=jax.ShapeDtypeStruct((seq, bsz, D), jnp.float32),
        scratch_types=[
            pltpu.VMEM((seq, bpw), jnp.int32),
            pltpu.VMEM((4, bpw, D), jnp.float32),
            pltpu.VMEM_SHARED((V, D), jnp.float32),
            pltpu.SemaphoreType.DMA((4,)),
            pltpu.SemaphoreType.DMA((4,)),
        ],
    )
    def k(idx_hbm, table_hbm, out_hbm, idx_v, rows_v, table_sh, gsem, wsem):
        sid = lax.axis_index("s")
        wid = sid * nc + lax.axis_index("c")
        base = wid * bpw
        pltpu.sync_copy(table_hbm.at[pl.ds(sid * v_per_tile, v_per_tile)],
                        table_sh.at[pl.ds(sid * v_per_tile, v_per_tile)])
        pltpu.sync_copy(idx_hbm.at[:, pl.ds(base, bpw)], idx_v)
        plsc.subcore_barrier()

        def gather(j, p):
            return pltpu.make_async_copy(
                table_sh.at[idx_v.at[j]], rows_v.at[p], gsem.at[p])

        def write(j, p):
            return pltpu.make_async_copy(
                rows_v.at[p], out_hbm.at[j, pl.ds(base, bpw)], wsem.at[p])

        gather(0, 0).start()
        gather(1, 1).start()

        def body(j, carry):
            p = j % 4
            q = (j + 2) % 4

            @pl.when(j >= 2)
            def _():
                write(j - 2, q).wait()

            @pl.when(j + 2 < seq)
            def _():
                gather(j + 2, q).start()

            gather(j, p).wait()
            write(j, p).start()
            return carry

        lax.fori_loop(0, seq, body, 0)
        write(seq - 2, (seq - 2) % 4).wait()
        write(seq - 1, (seq - 1) % 4).wait()

    return k


def kernel(indices, weights):
    bsz, seq = indices.shape
    V, D = weights.shape
    info = plsc.get_sparse_core_info()
    nc, ns = info.num_cores, info.num_subcores
    nw = nc * ns
    idxT = indices.astype(jnp.int32).T
    pad = (-bsz) % nw
    if pad:
        idxT = jnp.concatenate(
            [idxT, jnp.zeros((seq, pad), jnp.int32)], axis=1)
    bpw = (bsz + pad) // nw
    out = _make_gather(nw, nc, seq, bpw, D, V, bsz + pad)(idxT, weights)
    return jnp.transpose(out, (1, 0, 2))[:bsz]


# speedup vs baseline: 16.0929x; 1.0140x over previous
import functools

import jax
import jax.numpy as jnp
from jax import lax
from jax.experimental import pallas as pl
from jax.experimental.pallas import tpu as pltpu
from jax.experimental.pallas import tpu_sc as plsc


def _make_gather(nw, nc, seq, bpw, D, V, bsz):
    ns = nw // nc
    v_per_tile = V // ns
    mesh = plsc.VectorSubcoreMesh(core_axis_name="c", subcore_axis_name="s")

    @functools.partial(
        pl.kernel,
        mesh=mesh,
        out_type=jax.ShapeDtypeStruct((seq, bsz, D), jnp.float32),
        scratch_types=[
            pltpu.VMEM((seq, bpw), jnp.int32),
            pltpu.VMEM((6, bpw, D), jnp.float32),
            pltpu.VMEM_SHARED((V, D), jnp.float32),
            pltpu.SemaphoreType.DMA((6,)),
            pltpu.SemaphoreType.DMA((6,)),
        ],
    )
    def k(idx_hbm, table_hbm, out_hbm, idx_v, rows_v, table_sh, gsem, wsem):
        sid = lax.axis_index("s")
        wid = sid * nc + lax.axis_index("c")
        base = wid * bpw
        tstage = pltpu.make_async_copy(
            table_hbm.at[pl.ds(sid * v_per_tile, v_per_tile)],
            table_sh.at[pl.ds(sid * v_per_tile, v_per_tile)], gsem.at[2])
        istage = pltpu.make_async_copy(
            idx_hbm.at[:, pl.ds(base, bpw)], idx_v, gsem.at[3])
        tstage.start()
        istage.start()
        istage.wait()
        tstage.wait()
        plsc.subcore_barrier()

        def gather(j, p):
            return pltpu.make_async_copy(
                table_sh.at[idx_v.at[j]], rows_v.at[p], gsem.at[p])

        def write(j, p):
            return pltpu.make_async_copy(
                rows_v.at[p], out_hbm.at[j, pl.ds(base, bpw)], wsem.at[p])

        gather(0, 0).start()
        gather(1, 1).start()

        def body(j, carry):
            p = j % 6
            q = (j + 2) % 6

            @pl.when(j >= 4)
            def _():
                write(j - 4, q).wait()

            @pl.when(j + 2 < seq)
            def _():
                gather(j + 2, q).start()

            gather(j, p).wait()
            write(j, p).start()
            return carry

        lax.fori_loop(0, seq, body, 0)
        for t in range(4):
            j = seq - 4 + t
            write(j, j % 6).wait()

    return k


def kernel(indices, weights):
    bsz, seq = indices.shape
    V, D = weights.shape
    info = plsc.get_sparse_core_info()
    nc, ns = info.num_cores, info.num_subcores
    nw = nc * ns
    idxT = indices.astype(jnp.int32).T
    pad = (-bsz) % nw
    if pad:
        idxT = jnp.concatenate(
            [idxT, jnp.zeros((seq, pad), jnp.int32)], axis=1)
    bpw = (bsz + pad) // nw
    out = _make_gather(nw, nc, seq, bpw, D, V, bsz + pad)(idxT, weights)
    return jnp.transpose(out, (1, 0, 2))[:bsz]
